# baseline (speedup 1.0000x reference)
import functools
import math

import jax
import jax.numpy as jnp
from jax import lax
from jax.experimental import pallas as pl
from jax.experimental.pallas import tpu as pltpu
from jax.experimental.pallas import tpu_sc as plsc

SCALE = 4.0
THRESHOLD = 0.2
_LOGIT_T = math.log(THRESHOLD / (1.0 - THRESHOLD))

N_INST, H, W = 64, 512, 512
NC, NS, L = 2, 16, 16
NW = NC * NS

NSC = 64
NTC = N_INST - NSC
IPW = NSC // NW
CHUNK_ROWS = 64
CPI = H // CHUNK_ROWS
PAIRS = IPW * CPI // 2


def _masses_kernel(logits_hbm, out_hbm, buf0, buf1, out_v, sem0, sem1):
    wid = lax.axis_index("s") * NC + lax.axis_index("c")
    worker_row0 = (NTC + wid * IPW) * H

    bufs = (buf0, buf1)
    sems = (sem0, sem1)

    def start(g, b):
        pltpu.async_copy(
            logits_hbm.at[pl.ds(worker_row0 + g * CHUNK_ROWS, CHUNK_ROWS)],
            bufs[b],
            sems[b],
        )

    def wait(b):
        pltpu.make_async_copy(
            logits_hbm.at[pl.ds(0, CHUNK_ROWS)], bufs[b], sems[b]
        ).wait()

    lane_x0 = lax.iota(jnp.int32, L).astype(jnp.float32)
    zero_v = jnp.zeros((L,), jnp.float32)
    PACK = float(1 << 18)
    INV_PACK = 1.0 / PACK
    NACC = 16
    GROUPS = W // L // NACC

    def chunk_body(buf, row0, cnt, sx, sy):
        def row_body(r, carry):
            cnt, sx, sy = carry
            s = [None] * NACC
            xv = [lane_x0 + (PACK + 16.0 * k) for k in range(NACC)]
            for g in range(GROUPS):
                for k in range(NACC):
                    v = buf[r, pl.ds((g * NACC + k) * L, L)]
                    m = v > _LOGIT_T
                    t = jnp.where(m, xv[k], zero_v)
                    s[k] = t if g == 0 else s[k] + t
                if g + 1 < GROUPS:
                    xv = [x + float(NACC * L) for x in xv]
            while len(s) > 1:
                s = [a + b for a, b in zip(s[::2], s[1::2])]
            stot = s[0]
            s = [stot]
            rcf = (stot * INV_PACK).astype(jnp.int32).astype(jnp.float32)
            rx = stot - rcf * PACK
            y = (row0 + r).astype(jnp.float32)
            return (cnt + rcf, sx + rx, sy + y * rcf)

        return lax.fori_loop(0, CHUNK_ROWS, row_body, (cnt, sx, sy))

    start(0, 0)
    start(1, 1)

    def pair_body(p, carry):
        cnt, sx, sy = carry
        row0 = (2 * p) % CPI * CHUNK_ROWS
        for b in range(2):
            wait(b)

            @pl.when(p + 1 < PAIRS)
            def _():
                start(2 * p + 2 + b, b)

            cnt, sx, sy = chunk_body(bufs[b], row0 + b * CHUNK_ROWS,
                                     cnt, sx, sy)

        flush = (2 * p + 2) % CPI == 0

        @pl.when(flush)
        def _():
            off = (2 * p + 2) // CPI - 1
            for k, vec in enumerate((cnt, sx, sy)):
                out_v[pl.ds(off * (3 * L) + k * L, L)] = vec

        keep = jnp.where(flush, 0.0, 1.0)
        return (cnt * keep, sx * keep, sy * keep)

    lax.fori_loop(0, PAIRS, pair_body, (zero_v, zero_v, zero_v))
    pltpu.sync_copy(out_v, out_hbm.at[pl.ds(wid * (IPW * 3 * L), IPW * 3 * L)])


_masses = functools.partial(
    pl.kernel,
    out_type=jax.ShapeDtypeStruct((NW * IPW * 3 * L,), jnp.float32),
    mesh=plsc.VectorSubcoreMesh(core_axis_name="c", subcore_axis_name="s"),
    scratch_types=[
        pltpu.VMEM((CHUNK_ROWS, W), jnp.float32),
        pltpu.VMEM((CHUNK_ROWS, W), jnp.float32),
        pltpu.VMEM((IPW * 3 * L,), jnp.float32),
        pltpu.SemaphoreType.DMA,
        pltpu.SemaphoreType.DMA,
    ],
    compiler_params=pltpu.CompilerParams(use_tc_tiling_on_sc=True,
                                         skip_device_barrier=True,
                                         vmem_limit_bytes=4 * 1024 * 1024),
)(_masses_kernel)


TC_BLOCK_I = 8


def _tc_body(x_ref, o_ref):
    col = lax.broadcasted_iota(jnp.int32, (W, 128), 1)
    xf = lax.broadcasted_iota(jnp.int32, (W, 128), 0).astype(jnp.float32)
    xw = jnp.where(col == 0, xf, jnp.where(col == 1, 1.0, 0.0))
    r8 = lax.broadcasted_iota(jnp.int32, (8, H), 0)
    yf = lax.broadcasted_iota(jnp.int32, (8, H), 1).astype(jnp.float32)
    yw = jnp.where(r8 == 0, 1.0, jnp.where(r8 == 1, yf, 0.0))
    for i in range(TC_BLOCK_I):
        x = x_ref[pl.ds(i * H, H), :]
        m = jnp.where(x > _LOGIT_T, 1.0, 0.0)
        t1 = jnp.dot(m, xw, preferred_element_type=jnp.float32)
        o_ref[i] = jnp.dot(yw, t1, preferred_element_type=jnp.float32)


_tc_masses = pl.pallas_call(
    _tc_body,
    grid=(max(NTC, TC_BLOCK_I) // TC_BLOCK_I,),
    in_specs=[pl.BlockSpec((TC_BLOCK_I * H, W), lambda i: (i, 0))],
    out_specs=pl.BlockSpec((TC_BLOCK_I, 8, 128), lambda i: (i, 0, 0)),
    out_shape=jax.ShapeDtypeStruct((max(NTC, TC_BLOCK_I), 8, 128),
                                   jnp.float32),
    compiler_params=pltpu.CompilerParams(skip_device_barrier=True,
                                         vmem_limit_bytes=40 * 1024 * 1024),
)


def _inv3(K):
    a, b, c = K[0, 0], K[0, 1], K[0, 2]
    d, e, f = K[1, 0], K[1, 1], K[1, 2]
    g, h, i = K[2, 0], K[2, 1], K[2, 2]
    A = e * i - f * h
    B = c * h - b * i
    C = b * f - c * e
    D = f * g - d * i
    E = a * i - c * g
    F = c * d - a * f
    G = d * h - e * g
    Hc = b * g - a * h
    I = a * e - b * d
    det = a * A + b * D + c * G
    return jnp.array([[A, B, C], [D, E, F], [G, Hc, I]]) / det


def kernel(logits, mean_depths, K):
    n = logits.shape[0]
    flat = logits.reshape(n * H, W)
    sc = _masses(flat).reshape(NSC, 3, L).sum(axis=-1)
    if NTC:
        tcout = _tc_masses(flat)
        tc = jnp.stack([tcout[:, 0, 1], tcout[:, 0, 0], tcout[:, 1, 1]],
                       axis=1)
        mass = jnp.concatenate([tc, sc], axis=0)
    else:
        mass = sc
    counts, sum_x, sum_y = mass[:, 0], mass[:, 1], mass[:, 2]
    denom = jnp.maximum(counts, 1.0)
    mean_x = sum_x / denom * SCALE
    mean_y = sum_y / denom * SCALE
    ones = jnp.ones((n,), jnp.float32)
    xy1 = jnp.stack([mean_x, mean_y, ones], axis=1)
    return (xy1 @ _inv3(K).T) * mean_depths


# speedup vs baseline: 1.0457x; 1.0457x over previous
import functools
import math

import jax
import jax.numpy as jnp
from jax import lax
from jax.experimental import pallas as pl
from jax.experimental.pallas import tpu as pltpu
from jax.experimental.pallas import tpu_sc as plsc

SCALE = 4.0
THRESHOLD = 0.2
_LOGIT_T = math.log(THRESHOLD / (1.0 - THRESHOLD))

N_INST, H, W = 64, 512, 512
NC, NS, L = 2, 16, 16
NW = NC * NS

NSC = 64
NTC = N_INST - NSC
IPW = NSC // NW
CHUNK_ROWS = 64
CPI = H // CHUNK_ROWS
PAIRS = IPW * CPI // 2


def _masses_kernel(logits_hbm, out_hbm, buf0, buf1, out_v, sem0, sem1):
    wid = lax.axis_index("s") * NC + lax.axis_index("c")
    worker_row0 = (NTC + wid * IPW) * H

    bufs = (buf0, buf1)
    sems = (sem0, sem1)

    def start(g, b):
        pltpu.async_copy(
            logits_hbm.at[pl.ds(worker_row0 + g * CHUNK_ROWS, CHUNK_ROWS)],
            bufs[b],
            sems[b],
        )

    def wait(b):
        pltpu.make_async_copy(
            logits_hbm.at[pl.ds(0, CHUNK_ROWS)], bufs[b], sems[b]
        ).wait()

    lane_x0 = lax.iota(jnp.int32, L).astype(jnp.float32)
    zero_v = jnp.zeros((L,), jnp.float32)
    PACK = float(1 << 18)
    INV_PACK = 1.0 / PACK
    NACC = 8
    GROUPS = W // L // NACC

    def chunk_body(buf, row0, cnt, sx, sy):
        def row_body(r, carry):
            cnt, sx, sy = carry
            s = [None] * NACC
            xv = [lane_x0 + (PACK + 16.0 * k) for k in range(NACC)]
            for g in range(GROUPS):
                for k in range(NACC):
                    v = buf[r, pl.ds((g * NACC + k) * L, L)]
                    m = v > _LOGIT_T
                    t = jnp.where(m, xv[k], zero_v)
                    s[k] = t if g == 0 else s[k] + t
                if g + 1 < GROUPS:
                    xv = [x + float(NACC * L) for x in xv]
            while len(s) > 1:
                s = [a + b for a, b in zip(s[::2], s[1::2])]
            stot = s[0]
            s = [stot]
            rcf = (stot * INV_PACK).astype(jnp.int32).astype(jnp.float32)
            rx = stot - rcf * PACK
            y = (row0 + r).astype(jnp.float32)
            return (cnt + rcf, sx + rx, sy + y * rcf)

        return lax.fori_loop(0, CHUNK_ROWS, row_body, (cnt, sx, sy))

    start(0, 0)
    start(1, 1)

    def pair_body(p, carry):
        cnt, sx, sy = carry
        row0 = (2 * p) % CPI * CHUNK_ROWS
        for b in range(2):
            wait(b)

            @pl.when(p + 1 < PAIRS)
            def _():
                start(2 * p + 2 + b, b)

            cnt, sx, sy = chunk_body(bufs[b], row0 + b * CHUNK_ROWS,
                                     cnt, sx, sy)

        flush = (2 * p + 2) % CPI == 0

        @pl.when(flush)
        def _():
            off = (2 * p + 2) // CPI - 1
            for k, vec in enumerate((cnt, sx, sy)):
                out_v[pl.ds(off * (3 * L) + k * L, L)] = vec

        keep = jnp.where(flush, 0.0, 1.0)
        return (cnt * keep, sx * keep, sy * keep)

    lax.fori_loop(0, PAIRS, pair_body, (zero_v, zero_v, zero_v))
    pltpu.sync_copy(out_v, out_hbm.at[pl.ds(wid * (IPW * 3 * L), IPW * 3 * L)])


_masses = functools.partial(
    pl.kernel,
    out_type=jax.ShapeDtypeStruct((NW * IPW * 3 * L,), jnp.float32),
    mesh=plsc.VectorSubcoreMesh(core_axis_name="c", subcore_axis_name="s"),
    scratch_types=[
        pltpu.VMEM((CHUNK_ROWS, W), jnp.float32),
        pltpu.VMEM((CHUNK_ROWS, W), jnp.float32),
        pltpu.VMEM((IPW * 3 * L,), jnp.float32),
        pltpu.SemaphoreType.DMA,
        pltpu.SemaphoreType.DMA,
    ],
    compiler_params=pltpu.CompilerParams(use_tc_tiling_on_sc=True,
                                         skip_device_barrier=True,
                                         vmem_limit_bytes=4 * 1024 * 1024),
)(_masses_kernel)


TC_BLOCK_I = 8


def _tc_body(x_ref, o_ref):
    col = lax.broadcasted_iota(jnp.int32, (W, 128), 1)
    xf = lax.broadcasted_iota(jnp.int32, (W, 128), 0).astype(jnp.float32)
    xw = jnp.where(col == 0, xf, jnp.where(col == 1, 1.0, 0.0))
    r8 = lax.broadcasted_iota(jnp.int32, (8, H), 0)
    yf = lax.broadcasted_iota(jnp.int32, (8, H), 1).astype(jnp.float32)
    yw = jnp.where(r8 == 0, 1.0, jnp.where(r8 == 1, yf, 0.0))
    for i in range(TC_BLOCK_I):
        x = x_ref[pl.ds(i * H, H), :]
        m = jnp.where(x > _LOGIT_T, 1.0, 0.0)
        t1 = jnp.dot(m, xw, preferred_element_type=jnp.float32)
        o_ref[i] = jnp.dot(yw, t1, preferred_element_type=jnp.float32)


_tc_masses = pl.pallas_call(
    _tc_body,
    grid=(max(NTC, TC_BLOCK_I) // TC_BLOCK_I,),
    in_specs=[pl.BlockSpec((TC_BLOCK_I * H, W), lambda i: (i, 0))],
    out_specs=pl.BlockSpec((TC_BLOCK_I, 8, 128), lambda i: (i, 0, 0)),
    out_shape=jax.ShapeDtypeStruct((max(NTC, TC_BLOCK_I), 8, 128),
                                   jnp.float32),
    compiler_params=pltpu.CompilerParams(skip_device_barrier=True,
                                         vmem_limit_bytes=40 * 1024 * 1024),
)


def _inv3(K):
    a, b, c = K[0, 0], K[0, 1], K[0, 2]
    d, e, f = K[1, 0], K[1, 1], K[1, 2]
    g, h, i = K[2, 0], K[2, 1], K[2, 2]
    A = e * i - f * h
    B = c * h - b * i
    C = b * f - c * e
    D = f * g - d * i
    E = a * i - c * g
    F = c * d - a * f
    G = d * h - e * g
    Hc = b * g - a * h
    I = a * e - b * d
    det = a * A + b * D + c * G
    return jnp.array([[A, B, C], [D, E, F], [G, Hc, I]]) / det


def kernel(logits, mean_depths, K):
    n = logits.shape[0]
    flat = logits.reshape(n * H, W)
    sc = _masses(flat).reshape(NSC, 3, L).sum(axis=-1)
    if NTC:
        tcout = _tc_masses(flat)
        tc = jnp.stack([tcout[:, 0, 1], tcout[:, 0, 0], tcout[:, 1, 1]],
                       axis=1)
        mass = jnp.concatenate([tc, sc], axis=0)
    else:
        mass = sc
    counts, sum_x, sum_y = mass[:, 0], mass[:, 1], mass[:, 2]
    denom = jnp.maximum(counts, 1.0)
    mean_x = sum_x / denom * SCALE
    mean_y = sum_y / denom * SCALE
    ones = jnp.ones((n,), jnp.float32)
    xy1 = jnp.stack([mean_x, mean_y, ones], axis=1)
    return (xy1 @ _inv3(K).T) * mean_depths
